# baseline (speedup 1.0000x reference)
import jax
import jax.numpy as jnp
from jax.experimental import pallas as pl

K = 8192
D = 256
BN = 512
HALF = 4096
SL = 128


def _vq_kernel(x_ref, cb_ref, esq_ref, out_ref):
    x = x_ref[...]
    z_sq = jnp.sum(x * x, axis=1, keepdims=True)
    x2 = -2.0 * x
    lane = jax.lax.broadcasted_iota(jnp.int32, (BN, SL), 1).astype(jnp.float32)

    def half(base):
        cb = cb_ref[pl.ds(base, HALF), :]
        dot = jax.lax.dot_general(
            x2, cb, (((1,), (1,)), ((), ())),
            preferred_element_type=jnp.float32)
        dists = (z_sq + dot) + esq_ref[:, pl.ds(base, HALF)]
        val = jnp.full((BN, SL), jnp.inf, jnp.float32)
        cid = jnp.zeros((BN, SL), jnp.float32)
        for c in range(HALF // SL):
            d = dists[:, c * SL:(c + 1) * SL]
            lt = d < val
            val = jnp.where(lt, d, val)
            cid = jnp.where(lt, jnp.float32(c), cid)
        w = jnp.min(val, axis=1, keepdims=True)
        idx = cid * jnp.float32(SL) + lane + jnp.float32(base)
        li = jnp.min(jnp.where(val == w, idx, jnp.float32(K)),
                     axis=1, keepdims=True)
        return w, li

    w0, i0 = half(0)
    w1, i1 = half(HALF)
    carry = w0.astype(jnp.bfloat16).astype(jnp.float32)
    out_ref[...] = jnp.where(w1 < carry, i1, i0).astype(jnp.int32)


def kernel(z_e_x, codebook):
    B, Dd, H, W = z_e_x.shape
    flat = jnp.transpose(z_e_x, (0, 2, 3, 1)).reshape(-1, Dd)
    N = flat.shape[0]
    e_sq = jnp.sum(codebook * codebook, axis=1)[None, :]

    out = pl.pallas_call(
        _vq_kernel,
        grid=(N // BN,),
        in_specs=[
            pl.BlockSpec((BN, D), lambda n: (n, 0)),
            pl.BlockSpec((K, D), lambda n: (0, 0)),
            pl.BlockSpec((1, K), lambda n: (0, 0)),
        ],
        out_specs=pl.BlockSpec((BN, 1), lambda n: (n, 0)),
        out_shape=jax.ShapeDtypeStruct((N, 1), jnp.int32),
    )(flat, codebook, e_sq)
    return out[:, 0].reshape(B, H, W)


# speedup vs baseline: 1.0311x; 1.0311x over previous
import jax
import jax.numpy as jnp
from jax.experimental import pallas as pl
from jax.experimental.pallas import tpu as pltpu

K = 8192
D = 256
BN = 1024
HALF = 4096
SL = 128


def _vq_kernel(x_ref, cb_ref, out_ref, esq_ref):
    @pl.when(pl.program_id(0) == 0)
    def _init_esq():
        cbv = cb_ref[...]
        esq_ref[...] = jnp.sum(cbv * cbv, axis=1)[None, :]

    x = x_ref[...]
    z_sq = jnp.sum(x * x, axis=1, keepdims=True)
    x2 = -2.0 * x
    lane = jax.lax.broadcasted_iota(jnp.int32, (BN, SL), 1).astype(jnp.float32)

    def half(base):
        cb = cb_ref[pl.ds(base, HALF), :]
        dot = jax.lax.dot_general(
            x2, cb, (((1,), (1,)), ((), ())),
            preferred_element_type=jnp.float32)
        dists = (z_sq + dot) + esq_ref[:, pl.ds(base, HALF)]
        val = jnp.full((BN, SL), jnp.inf, jnp.float32)
        cid = jnp.zeros((BN, SL), jnp.float32)
        for c in range(HALF // SL):
            d = dists[:, c * SL:(c + 1) * SL]
            lt = d < val
            val = jnp.where(lt, d, val)
            cid = jnp.where(lt, jnp.float32(c), cid)
        w = jnp.min(val, axis=1, keepdims=True)
        idx = cid * jnp.float32(SL) + lane + jnp.float32(base)
        li = jnp.min(jnp.where(val == w, idx, jnp.float32(K)),
                     axis=1, keepdims=True)
        return w, li

    w0, i0 = half(0)
    w1, i1 = half(HALF)
    carry = w0.astype(jnp.bfloat16).astype(jnp.float32)
    out_ref[...] = jnp.where(w1 < carry, i1, i0).astype(jnp.int32)


def kernel(z_e_x, codebook):
    B, Dd, H, W = z_e_x.shape
    flat = jnp.transpose(z_e_x, (0, 2, 3, 1)).reshape(-1, Dd)
    N = flat.shape[0]

    out = pl.pallas_call(
        _vq_kernel,
        grid=(N // BN,),
        in_specs=[
            pl.BlockSpec((BN, D), lambda n: (n, 0)),
            pl.BlockSpec((K, D), lambda n: (0, 0)),
        ],
        out_specs=pl.BlockSpec((BN, 1), lambda n: (n, 0)),
        out_shape=jax.ShapeDtypeStruct((N, 1), jnp.int32),
        scratch_shapes=[pltpu.VMEM((1, K), jnp.float32)],
    )(flat, codebook)
    return out[:, 0].reshape(B, H, W)
